# baseline (speedup 1.0000x reference)
import jax
import jax.numpy as jnp
from jax import lax
from jax.experimental import pallas as pl
from jax.experimental.pallas import tpu as pltpu
from jax.experimental.pallas import tpu_sc as plsc

_NC = 2
_NS = 16
_NW = _NC * _NS
_L = 16

_B = 16
_CIN = 10
_COUT = 8
_H = 512
_W = 512

_R = 8
_CPP = _H // _R
_NCHUNK = _B * _CPP
_CPW = _NCHUNK // _NW
_NBUF = 3


def _mag(g8, g9):
    s = g8 * g8 + g9 * g9
    si = lax.bitcast_convert_type(s, jnp.int32)
    yi = jnp.int32(0x5F3759DF) - lax.shift_right_arithmetic(si, jnp.int32(1))
    y = lax.bitcast_convert_type(yi, jnp.float32)
    hs = 0.5 * s
    for _ in range(1):
        y = y * (1.5 - hs * y * y)
    return s * y


def _sc_body(x_hbm, o_hbm, bufs, isems, osems):
    wid = lax.axis_index("s") * _NC + lax.axis_index("c")
    base = wid * _CPW

    def start_in(slot, chunk):
        b = chunk // _CPP
        r0 = (chunk % _CPP) * _R
        for c in range(_CIN):
            pltpu.async_copy(x_hbm.at[b, c, pl.ds(r0, _R), :], bufs[slot].at[c],
                             isems[slot])

    def wait_in(slot):
        for c in range(_CIN):
            pltpu.make_async_copy(x_hbm.at[0, 0, pl.ds(0, _R), :], bufs[slot].at[c],
                                  isems[slot]).wait()

    def start_out(slot, chunk):
        b = chunk // _CPP
        r0 = (chunk % _CPP) * _R
        for c in range(_COUT):
            pltpu.async_copy(bufs[slot].at[c], o_hbm.at[b, c, pl.ds(r0, _R), :],
                             osems[slot])

    def wait_out(slot):
        for c in range(_COUT):
            pltpu.make_async_copy(bufs[slot].at[c], o_hbm.at[0, 0, pl.ds(0, _R), :],
                                  osems[slot]).wait()

    lanes = lax.iota(jnp.int32, _L)
    zero = jnp.zeros((_L,), jnp.float32)

    def compute(slot):
        ib = bufs[slot]

        def px_body(j, carry):
            col = pl.ds(pl.multiple_of(j * _L, _L), _L)
            col_idx = j * _L + lanes
            for r in range(_R):
                vals = [ib[c, r, col] for c in range(_CIN)]
                mag = _mag(vals[8], vals[9])
                best = vals[0]
                bi = jnp.zeros((_L,), jnp.int32)
                for c in range(1, _COUT):
                    gt = vals[c] > best
                    best = jnp.where(gt, vals[c], best)
                    bi = jnp.where(gt, jnp.int32(c), bi)
                for c in range(_COUT):
                    ib[c, r, col] = zero
                plsc.store_scatter(
                    ib, [bi, jnp.full((_L,), r, jnp.int32), col_idx], mag)
            return carry

        lax.fori_loop(0, _W // _L, px_body, 0)

    start_in(0, base + 0)
    start_in(1, base + 1)

    _MAIN = _CPW - _NBUF + 1

    def step(t, carry):
        for s in range(_NBUF):
            chunk = base + t + s
            wait_in(s)
            compute(s)
            start_out(s, chunk)
            rs = (s + 2) % _NBUF

            @pl.when(t + s >= 1)
            def _():
                wait_out(rs)

            start_in(rs, chunk + 2)
        return carry

    lax.fori_loop(0, _MAIN // _NBUF, lambda t, c: step(t * _NBUF, c), 0)
    for s in range(_CPW - _MAIN):
        chunk = base + _MAIN + s
        wait_in(s)
        compute(s)
        start_out(s, chunk)
        wait_out((s + 2) % _NBUF)
    wait_out((_CPW - 1) % _NBUF)


_sc_kernel = pl.kernel(
    _sc_body,
    out_type=jax.ShapeDtypeStruct((_B, _COUT, _H, _W), jnp.float32),
    mesh=plsc.VectorSubcoreMesh(core_axis_name="c", subcore_axis_name="s"),
    compiler_params=pltpu.CompilerParams(use_tc_tiling_on_sc=True,
                                         needs_layout_passes=False),
    scratch_types=[
        [pltpu.VMEM((_CIN, _R, _W), jnp.float32) for _ in range(_NBUF)],
        [pltpu.SemaphoreType.DMA for _ in range(_NBUF)],
        [pltpu.SemaphoreType.DMA for _ in range(_NBUF)],
    ],
)


def kernel(x):
    return _sc_kernel(x)


# speedup vs baseline: 1.1032x; 1.1032x over previous
import jax
import jax.numpy as jnp
from jax import lax
from jax.experimental import pallas as pl
from jax.experimental.pallas import tpu as pltpu
from jax.experimental.pallas import tpu_sc as plsc

_NC = 2
_NS = 16
_NW = _NC * _NS
_L = 16

_B = 16
_CIN = 10
_COUT = 8
_H = 512
_W = 512

_R = 8
_CPP = _H // _R
_NCHUNK = _B * _CPP
_CPW = _NCHUNK // _NW
_NBUF = 3


def _mag(g8, g9):
    s = g8 * g8 + g9 * g9
    si = lax.bitcast_convert_type(s, jnp.int32)
    yi = jnp.int32(0x5F3759DF) - lax.shift_right_arithmetic(si, jnp.int32(1))
    y = lax.bitcast_convert_type(yi, jnp.float32)
    hs = 0.5 * s
    for _ in range(1):
        y = y * (1.5 - hs * y * y)
    return s * y


def _sc_body(x_hbm, o_hbm, bufs, isems, osems):
    wid = lax.axis_index("s") * _NC + lax.axis_index("c")
    base = wid * _CPW

    def start_in(slot, chunk):
        b = chunk // _CPP
        r0 = (chunk % _CPP) * _R
        for c in range(_CIN):
            pltpu.async_copy(x_hbm.at[b, c, pl.ds(r0, _R), :], bufs[slot][c],
                             isems[slot])

    def wait_in(slot):
        for c in range(_CIN):
            pltpu.make_async_copy(x_hbm.at[0, 0, pl.ds(0, _R), :], bufs[slot][c],
                                  isems[slot]).wait()

    def start_out(slot, chunk):
        b = chunk // _CPP
        r0 = (chunk % _CPP) * _R
        for c in range(_COUT):
            pltpu.async_copy(bufs[slot][c], o_hbm.at[b, c, pl.ds(r0, _R), :],
                             osems[slot])

    def wait_out(slot):
        for c in range(_COUT):
            pltpu.make_async_copy(bufs[slot][c], o_hbm.at[0, 0, pl.ds(0, _R), :],
                                  osems[slot]).wait()

    def compute(slot):
        ib = bufs[slot]

        def px_body(j, carry):
            col = pl.ds(pl.multiple_of(j * _L, _L), _L)
            for r in range(_R):
                vals = [ib[c][r, col] for c in range(_CIN)]
                mag = _mag(vals[8], vals[9])
                best = vals[0]
                bi = jnp.zeros((_L,), jnp.int32)
                for c in range(1, _COUT):
                    gt = vals[c] > best
                    best = jnp.where(gt, vals[c], best)
                    bi = jnp.where(gt, jnp.int32(c), bi)
                zero = jnp.zeros((_L,), jnp.float32)
                for c in range(_COUT):
                    ib[c][r, col] = jnp.where(bi == jnp.int32(c), mag, zero)
            return carry

        lax.fori_loop(0, _W // _L, px_body, 0)

    start_in(0, base + 0)
    start_in(1, base + 1)

    _MAIN = _CPW - _NBUF + 1

    def step(t, carry):
        for s in range(_NBUF):
            chunk = base + t + s
            wait_in(s)
            compute(s)
            start_out(s, chunk)
            rs = (s + 2) % _NBUF

            @pl.when(t + s >= 1)
            def _():
                wait_out(rs)

            start_in(rs, chunk + 2)
        return carry

    lax.fori_loop(0, _MAIN // _NBUF, lambda t, c: step(t * _NBUF, c), 0)
    for s in range(_CPW - _MAIN):
        chunk = base + _MAIN + s
        wait_in(s)
        compute(s)
        start_out(s, chunk)
        wait_out((s + 2) % _NBUF)
    wait_out((_CPW - 1) % _NBUF)


_sc_kernel = pl.kernel(
    _sc_body,
    out_type=jax.ShapeDtypeStruct((_B, _COUT, _H, _W), jnp.float32),
    mesh=plsc.VectorSubcoreMesh(core_axis_name="c", subcore_axis_name="s"),
    compiler_params=pltpu.CompilerParams(use_tc_tiling_on_sc=True),
    scratch_types=[
        [[pltpu.VMEM((_R, _W), jnp.float32) for _ in range(_CIN)]
         for _ in range(_NBUF)],
        [pltpu.SemaphoreType.DMA for _ in range(_NBUF)],
        [pltpu.SemaphoreType.DMA for _ in range(_NBUF)],
    ],
)


def kernel(x):
    return _sc_kernel(x)
